# baseline (speedup 1.0000x reference)
import functools

import jax
import jax.numpy as jnp
from jax import lax
from jax.experimental import pallas as pl
from jax.experimental.pallas import tpu as pltpu
from jax.experimental.pallas import tpu_sc as plsc

N_DICT = 1000000
EMB = 64
BATCH = 16384

NC = 2
NS = 16
NW = NC * NS
BPW = BATCH // NW
RPW = 2 * BPW
NCHUNK = 8
CHUNK = RPW // NCHUNK
GROUPS = BPW // 16


def _body(x_hbm, table_hbm, w_hbm, out_hbm, idx_v, rows_v, w_v, out_v, sem):
    wid = lax.axis_index("s") * NC + lax.axis_index("c")

    pltpu.sync_copy(x_hbm.at[wid], idx_v)
    pltpu.sync_copy(w_hbm, w_v)

    copies = [
        pltpu.async_copy(
            table_hbm.at[idx_v.at[j]],
            rows_v.at[pl.ds(j * CHUNK, CHUNK)],
            sem,
        )
        for j in range(NCHUNK)
    ]
    for c in copies:
        c.wait()

    iota2 = lax.iota(jnp.int32, 16) * 2
    wvecs = [w_v[pl.ds(i * 16, 16)] for i in range(2 * EMB // 16)]
    bias = w_v[pl.ds(2 * EMB, 16)][0]

    def group(g, carry):
        r0 = g * 32 + iota2
        r1 = r0 + 1
        acc = jnp.zeros((16,), jnp.float32) + bias
        for k in range(EMB):
            ck = jnp.zeros((16,), jnp.int32) + k
            v0 = plsc.load_gather(rows_v, [r0, ck])
            v1 = plsc.load_gather(rows_v, [r1, ck])
            w0 = wvecs[k // 16][k % 16]
            w1 = wvecs[(EMB + k) // 16][k % 16]
            acc = acc + v0 * w0 + v1 * w1
        out_v[pl.ds(g * 16, 16)] = 1.0 / (1.0 + jnp.exp(-acc))
        return carry

    lax.fori_loop(0, GROUPS, group, 0)

    pltpu.sync_copy(out_v, out_hbm.at[pl.ds(wid * BPW, BPW)])


_sc_call = functools.partial(
    pl.kernel,
    out_type=jax.ShapeDtypeStruct((BATCH,), jnp.float32),
    scratch_types=[
        pltpu.VMEM((NCHUNK, CHUNK), jnp.int32),
        pltpu.VMEM((RPW, EMB), jnp.float32),
        pltpu.VMEM((144,), jnp.float32),
        pltpu.VMEM((BPW,), jnp.float32),
        pltpu.SemaphoreType.DMA,
    ],
    mesh=plsc.VectorSubcoreMesh(core_axis_name="c", subcore_axis_name="s"),
    compiler_params=pltpu.CompilerParams(
        needs_layout_passes=False, use_tc_tiling_on_sc=False
    ),
)(_body)


def kernel(x, table, W, b):
    x3 = x.astype(jnp.int32).reshape(NW, NCHUNK, CHUNK)
    wpad = jnp.zeros((144,), jnp.float32)
    wpad = wpad.at[: 2 * EMB].set(W[:, 0]).at[2 * EMB].set(b[0])
    out = _sc_call(x3, table, wpad)
    return out.reshape(BATCH, 1)


# speedup vs baseline: 6.5079x; 6.5079x over previous
import functools

import jax
import jax.numpy as jnp
from jax import lax
from jax.experimental import pallas as pl
from jax.experimental.pallas import tpu as pltpu
from jax.experimental.pallas import tpu_sc as plsc

N_DICT = 1000000
EMB = 64
BATCH = 16384


BLK = 32768
GRID = -(-N_DICT // BLK)


def _matvec_body(t_ref, w_ref, z0_ref, z1_ref):
    p = jnp.dot(w_ref[...], t_ref[...], preferred_element_type=jnp.float32)
    z0_ref[...] = p[0]
    z1_ref[...] = p[1]


_matvec = pl.pallas_call(
    _matvec_body,
    grid=(GRID,),
    in_specs=[
        pl.BlockSpec((EMB, BLK), lambda i: (0, i)),
        pl.BlockSpec((8, EMB), lambda i: (0, 0)),
    ],
    out_specs=[
        pl.BlockSpec((BLK,), lambda i: (i,)),
        pl.BlockSpec((BLK,), lambda i: (i,)),
    ],
    out_shape=[
        jax.ShapeDtypeStruct((N_DICT,), jnp.float32),
        jax.ShapeDtypeStruct((N_DICT,), jnp.float32),
    ],
    compiler_params=pltpu.CompilerParams(
        dimension_semantics=("arbitrary",),
    ),
)


NC = 2
NS = 16
NW = NC * NS
BPW = BATCH // NW
NCHUNK = 4
CHUNK = BPW // NCHUNK


def _gather_body(z0_hbm, z1_hbm, x0_hbm, x1_hbm, b_hbm, out_hbm,
                 idx0_v, idx1_v, g0_v, g1_v, b_v, out_v, sem):
    wid = lax.axis_index("s") * NC + lax.axis_index("c")
    base = wid * BPW

    pltpu.sync_copy(x0_hbm.at[pl.ds(base, BPW)], idx0_v)
    pltpu.sync_copy(x1_hbm.at[pl.ds(base, BPW)], idx1_v)
    pltpu.sync_copy(b_hbm, b_v)

    copies = []
    for j in range(NCHUNK):
        sl = pl.ds(j * CHUNK, CHUNK)
        copies.append(pltpu.async_copy(z0_hbm.at[idx0_v.at[sl]], g0_v.at[sl], sem))
        copies.append(pltpu.async_copy(z1_hbm.at[idx1_v.at[sl]], g1_v.at[sl], sem))
    for c in copies:
        c.wait()

    bias = b_v[...]
    for v in range(BPW // 16):
        sl = pl.ds(v * 16, 16)
        acc = g0_v[sl] + g1_v[sl] + bias
        out_v[sl] = 1.0 / (1.0 + jnp.exp(-acc))

    pltpu.sync_copy(out_v, out_hbm.at[pl.ds(base, BPW)])


_gather_call = functools.partial(
    pl.kernel,
    out_type=jax.ShapeDtypeStruct((BATCH,), jnp.float32),
    scratch_types=[
        pltpu.VMEM((BPW,), jnp.int32),
        pltpu.VMEM((BPW,), jnp.int32),
        pltpu.VMEM((BPW,), jnp.float32),
        pltpu.VMEM((BPW,), jnp.float32),
        pltpu.VMEM((16,), jnp.float32),
        pltpu.VMEM((BPW,), jnp.float32),
        pltpu.SemaphoreType.DMA,
    ],
    mesh=plsc.VectorSubcoreMesh(core_axis_name="c", subcore_axis_name="s"),
    compiler_params=pltpu.CompilerParams(
        needs_layout_passes=False, use_tc_tiling_on_sc=False
    ),
)(_gather_body)


def kernel(x, table, W, b):
    w8 = jnp.zeros((8, EMB), jnp.float32)
    w8 = w8.at[0].set(W[:EMB, 0]).at[1].set(W[EMB:, 0])
    z0, z1 = _matvec(table.T, w8)
    xi = x.astype(jnp.int32)
    bvec = jnp.full((16,), b[0], jnp.float32)
    out = _gather_call(z0, z1, xi[:, 0], xi[:, 1], bvec)
    return out.reshape(BATCH, 1)
